# Initial kernel scaffold; baseline (speedup 1.0000x reference)
import jax
import jax.numpy as jnp
from jax.experimental import pallas as pl


def kernel(x, shared_w1, shared_w2, routing_w1, routing_w2, router_w):
    raise NotImplementedError("write your pallas kernel here")


# speedup vs baseline: 3.3710x; 3.3710x over previous
import functools

import jax
import jax.numpy as jnp
from jax import lax
from jax.experimental import pallas as pl
from jax.experimental.pallas import tpu as pltpu
from jax.experimental.pallas import tpu_sc as plsc

_TOP_K = 2
_TILE_M = 128
_TILE_A = 256
_SQRT_HALF = 0.7071067811865476

_NC = 2
_NS = 16
_NW = _NC * _NS


def _gelu(h):
    return h * 0.5 * (1.0 + lax.erf(h * _SQRT_HALF))


def _shared_router_body(x_ref, w1_ref, w2_ref, rw_ref,
                        shared_ref, logits_ref, psum_ref):
    i = pl.program_id(0)
    xb = x_ref[...]
    h = _gelu(jnp.dot(xb, w1_ref[...], preferred_element_type=jnp.float32))
    shared_ref[...] = jnp.dot(h, w2_ref[...], preferred_element_type=jnp.float32)
    lg = jnp.dot(xb, rw_ref[...], preferred_element_type=jnp.float32)
    logits_ref[...] = lg
    m = jnp.max(lg, axis=-1, keepdims=True)
    p = jnp.exp(lg - m)
    p = p / jnp.sum(p, axis=-1, keepdims=True)

    @pl.when(i == 0)
    def _():
        psum_ref[...] = jnp.zeros_like(psum_ref)

    psum_ref[...] += jnp.sum(p, axis=0, keepdims=True)


def _shared_router(xf, w1c, w2c, rw):
    n, d = xf.shape
    e = rw.shape[1]
    hs = w1c.shape[1]
    return pl.pallas_call(
        _shared_router_body,
        grid=(n // _TILE_A,),
        in_specs=[
            pl.BlockSpec((_TILE_A, d), lambda i: (i, 0)),
            pl.BlockSpec((d, hs), lambda i: (0, 0)),
            pl.BlockSpec((hs, d), lambda i: (0, 0)),
            pl.BlockSpec((d, e), lambda i: (0, 0)),
        ],
        out_specs=[
            pl.BlockSpec((_TILE_A, d), lambda i: (i, 0)),
            pl.BlockSpec((_TILE_A, e), lambda i: (i, 0)),
            pl.BlockSpec((1, e), lambda i: (0, 0)),
        ],
        out_shape=[
            jax.ShapeDtypeStruct((n, d), jnp.float32),
            jax.ShapeDtypeStruct((n, e), jnp.float32),
            jax.ShapeDtypeStruct((1, e), jnp.float32),
        ],
    )(xf, w1c, w2c, rw)


def _make_dispatch_gather(pad, n, d):
    rpw = pad // _NW
    ch = 64
    nch = rpw // ch
    mesh = plsc.VectorSubcoreMesh(core_axis_name="c", subcore_axis_name="s")

    @functools.partial(
        pl.kernel, mesh=mesh,
        out_type=jax.ShapeDtypeStruct((pad, d), jnp.float32),
        scratch_types=[
            pltpu.VMEM((ch,), jnp.int32),
            pltpu.VMEM((ch, d), jnp.float32),
            pltpu.SemaphoreType.DMA,
        ],
    )
    def gather_k(idx_hbm, x_hbm, out_hbm, idx_v, rows_v, sem):
        wid = lax.axis_index("s") * _NC + lax.axis_index("c")
        for c in range(nch):
            base = pl.multiple_of(wid * rpw + c * ch, 8)
            pltpu.sync_copy(idx_hbm.at[pl.ds(base, ch)], idx_v)
            pltpu.async_copy(x_hbm.at[idx_v], rows_v, sem).wait()
            pltpu.sync_copy(rows_v, out_hbm.at[pl.ds(base, ch)])

    return gather_k


def _ffn_body(gid_ref, xg_ref, w1_ref, w2_ref, gate_ref, out_ref):
    xb = xg_ref[...]
    h = _gelu(jnp.dot(xb, w1_ref[0], preferred_element_type=jnp.float32))
    y = jnp.dot(h, w2_ref[0], preferred_element_type=jnp.float32)
    out_ref[...] = y * gate_ref[...]


def _grouped_ffn(gid, xg, rw1, rw2, gate_col):
    pad, d = xg.shape
    e, _, hr = rw1.shape
    nt = pad // _TILE_M
    grid_spec = pltpu.PrefetchScalarGridSpec(
        num_scalar_prefetch=1,
        grid=(nt,),
        in_specs=[
            pl.BlockSpec((_TILE_M, d), lambda m, g: (m, 0)),
            pl.BlockSpec((1, d, hr), lambda m, g: (g[m], 0, 0)),
            pl.BlockSpec((1, hr, d), lambda m, g: (g[m], 0, 0)),
            pl.BlockSpec((_TILE_M, 1), lambda m, g: (m, 0)),
        ],
        out_specs=pl.BlockSpec((_TILE_M, d), lambda m, g: (m, 0)),
    )
    return pl.pallas_call(
        _ffn_body,
        grid_spec=grid_spec,
        out_shape=jax.ShapeDtypeStruct((pad, d), jnp.float32),
    )(gid, xg, rw1, rw2, gate_col)


def _make_combine_gather(pad, n, d):
    tpw = n // _NW
    ch = 64
    nch = tpw // ch
    mesh = plsc.VectorSubcoreMesh(core_axis_name="c", subcore_axis_name="s")

    @functools.partial(
        pl.kernel, mesh=mesh,
        out_type=[
            jax.ShapeDtypeStruct((n, d), jnp.float32),
            jax.ShapeDtypeStruct((n, d), jnp.float32),
        ],
        scratch_types=[
            pltpu.VMEM((ch,), jnp.int32),
            pltpu.VMEM((ch, d), jnp.float32),
            pltpu.SemaphoreType.DMA,
        ],
    )
    def combine_k(pa_hbm, pb_hbm, yg_hbm, ya_hbm, yb_hbm, idx_v, rows_v, sem):
        wid = lax.axis_index("s") * _NC + lax.axis_index("c")
        for c in range(nch):
            base = pl.multiple_of(wid * tpw + c * ch, 8)
            pltpu.sync_copy(pa_hbm.at[pl.ds(base, ch)], idx_v)
            pltpu.async_copy(yg_hbm.at[idx_v], rows_v, sem).wait()
            pltpu.sync_copy(rows_v, ya_hbm.at[pl.ds(base, ch)])
            pltpu.sync_copy(pb_hbm.at[pl.ds(base, ch)], idx_v)
            pltpu.async_copy(yg_hbm.at[idx_v], rows_v, sem).wait()
            pltpu.sync_copy(rows_v, yb_hbm.at[pl.ds(base, ch)])

    return combine_k


def _add3_body(a_ref, b_ref, c_ref, o_ref):
    o_ref[...] = a_ref[...] + b_ref[...] + c_ref[...]


def _add3(a, b, c):
    n, d = a.shape
    spec = pl.BlockSpec((_TILE_A, d), lambda i: (i, 0))
    return pl.pallas_call(
        _add3_body,
        grid=(n // _TILE_A,),
        in_specs=[spec, spec, spec],
        out_specs=spec,
        out_shape=jax.ShapeDtypeStruct((n, d), jnp.float32),
    )(a, b, c)


def kernel(x, shared_w1, shared_w2, routing_w1, routing_w2, router_w):
    bv, tv, dv = x.shape
    n = bv * tv
    e = routing_w1.shape[0]
    pad = _TOP_K * n + e * _TILE_M
    nt = pad // _TILE_M
    xf = x.reshape(n, dv)

    w1c = jnp.concatenate([shared_w1[0], shared_w1[1]], axis=1)
    w2c = jnp.concatenate([shared_w2[0], shared_w2[1]], axis=0)

    shared, logits, psum = _shared_router(xf, w1c, w2c, router_w)

    top_vals, top_idx = lax.top_k(logits, _TOP_K)
    gates = jax.nn.softmax(top_vals, axis=-1)
    e_flat = top_idx.reshape(-1).astype(jnp.int32)
    g_flat = gates.reshape(-1)
    order = jnp.argsort(e_flat, stable=True).astype(jnp.int32)
    e_sorted = e_flat[order]
    counts = jnp.bincount(e_flat, length=e)
    start = jnp.concatenate(
        [jnp.zeros(1, jnp.int32), jnp.cumsum(counts)[:-1].astype(jnp.int32)])
    padded = ((counts + _TILE_M - 1) // _TILE_M) * _TILE_M
    astart = jnp.concatenate(
        [jnp.zeros(1, jnp.int32), jnp.cumsum(padded)[:-1].astype(jnp.int32)])
    rank = jnp.arange(_TOP_K * n, dtype=jnp.int32) - start[e_sorted]
    pos = astart[e_sorted] + rank
    src_tok = jnp.zeros(pad, jnp.int32).at[pos].set(order // _TOP_K)
    gate_arr = jnp.zeros(pad, jnp.float32).at[pos].set(g_flat[order])
    invpos = jnp.zeros(_TOP_K * n, jnp.int32).at[order].set(pos)
    pos_a = invpos[0::2]
    pos_b = invpos[1::2]
    gid = (jnp.searchsorted(astart, jnp.arange(nt) * _TILE_M, side='right') - 1
           ).astype(jnp.int32)

    xg = _make_dispatch_gather(pad, n, dv)(src_tok, xf)
    yg = _grouped_ffn(gid, xg, routing_w1, routing_w2, gate_arr[:, None])
    ya, yb = _make_combine_gather(pad, n, dv)(pos_a, pos_b, yg)
    out = _add3(shared, ya, yb)

    load = psum[0] / n
    balance_loss = e * jnp.sum(load * load)
    return out.reshape(bv, tv, dv), balance_loss
